# baseline (speedup 1.0000x reference)
import functools

import jax
import jax.numpy as jnp
from jax import lax
from jax.experimental import pallas as pl
from jax.experimental.pallas import tpu as pltpu
from jax.experimental.pallas import tpu_sc as plsc

E = 8
D = 1024
H = 512
T = 4096
N = 2 * T
BS = 256
NB = N // BS + E - 1
NP = NB * BS
NW = 32
TPW = T // NW
CH = 64


def _router_body(x_ref, wr_ref, cw_ref, pos_ref, bexp_ref,
                 a_ref, r_ref, g_ref, gx_ref):
    x = x_ref[...]
    logits = lax.dot_general(x, wr_ref[...], (((1,), (1,)), ((), ())),
                             preferred_element_type=jnp.float32)
    iota_e = lax.broadcasted_iota(jnp.int32, (T, E), 1)
    l0 = jnp.max(logits, axis=1, keepdims=True)
    e0 = jnp.min(jnp.where(logits == l0, iota_e, E), axis=1, keepdims=True)
    masked = jnp.where(iota_e == e0, -jnp.inf, logits)
    l1 = jnp.max(masked, axis=1, keepdims=True)
    e1 = jnp.min(jnp.where(masked == l1, iota_e, E), axis=1, keepdims=True)
    c0 = jax.nn.sigmoid(l0 - l1)
    cw_ref[0:T, :] = c0
    cw_ref[T:N, :] = 1.0 - c0
    a_ref[0:T, :] = (iota_e == e0).astype(jnp.float32)
    a_ref[T:N, :] = (iota_e == e1).astype(jnp.float32)

    tri = (lax.broadcasted_iota(jnp.int32, (128, 128), 0) >=
           lax.broadcasted_iota(jnp.int32, (128, 128), 1)
           ).astype(jnp.float32)
    ngrp = N // 128

    def loop1(g, _):
        blk = a_ref[pl.ds(g * 128, 128), :]
        pre = lax.dot_general(tri, blk, (((1,), (0,)), ((), ())),
                              preferred_element_type=jnp.float32)
        r_ref[pl.ds(g * 128, 128), :] = pre
        g_ref[pl.ds(g, 1), :] = pre[127:128, :]
        return 0
    lax.fori_loop(0, ngrp, loop1, 0)

    gmat = g_ref[...]
    e64 = (lax.broadcasted_iota(jnp.int32, (ngrp, ngrp), 1) <
           lax.broadcasted_iota(jnp.int32, (ngrp, ngrp), 0)
           ).astype(jnp.float32)
    gx_ref[...] = lax.dot_general(e64, gmat, (((1,), (0,)), ((), ())),
                                  preferred_element_type=jnp.float32)
    counts = jnp.sum(gmat, axis=0, keepdims=True)
    pad = ((counts.astype(jnp.int32) + BS - 1) // BS) * BS
    u = (lax.broadcasted_iota(jnp.int32, (E, E), 0) <
         lax.broadcasted_iota(jnp.int32, (E, E), 1)).astype(jnp.float32)
    off = lax.dot_general(pad.astype(jnp.float32), u,
                          (((1,), (0,)), ((), ())),
                          preferred_element_type=jnp.float32)

    def loop2(g, _):
        ab = a_ref[pl.ds(g * 128, 128), :]
        rb = r_ref[pl.ds(g * 128, 128), :]
        gx = gx_ref[pl.ds(g, 1), :]
        pm = ab * (rb - 1.0 + gx + off)
        p = jnp.sum(pm, axis=1, keepdims=True)
        pos_ref[pl.ds(g * 128, 128), :] = p.astype(jnp.int32)
        return 0
    lax.fori_loop(0, ngrp, loop2, 0)

    pst = (lax.broadcasted_iota(jnp.int32, (64, 1), 0) * BS).astype(
        jnp.float32)
    bexp_ref[...] = jnp.sum((off <= pst).astype(jnp.int32), axis=1,
                            keepdims=True) - 1


def _router(x_flat, Wr):
    return pl.pallas_call(
        _router_body,
        grid=(1,),
        in_specs=[
            pl.BlockSpec((T, D), lambda i: (0, 0)),
            pl.BlockSpec((E, D), lambda i: (0, 0)),
        ],
        out_specs=[
            pl.BlockSpec((N, 1), lambda i: (0, 0)),
            pl.BlockSpec((N, 1), lambda i: (0, 0)),
            pl.BlockSpec((64, 1), lambda i: (0, 0)),
        ],
        out_shape=[
            jax.ShapeDtypeStruct((N, 1), jnp.float32),
            jax.ShapeDtypeStruct((N, 1), jnp.int32),
            jax.ShapeDtypeStruct((64, 1), jnp.int32),
        ],
        scratch_shapes=[
            pltpu.VMEM((N, E), jnp.float32),
            pltpu.VMEM((N, E), jnp.float32),
            pltpu.VMEM((N // 128, E), jnp.float32),
            pltpu.VMEM((N // 128, E), jnp.float32),
        ],
    )(x_flat, Wr)


@functools.lru_cache(maxsize=None)
def _make_scatter():
    mesh = plsc.VectorSubcoreMesh(core_axis_name="c", subcore_axis_name="s")

    @functools.partial(
        pl.kernel, mesh=mesh,
        out_type=jax.ShapeDtypeStruct((NP, D), jnp.float32),
        scratch_types=[
            pltpu.VMEM((CH,), jnp.int32),
            pltpu.VMEM((CH, D), jnp.float32),
            pltpu.SemaphoreType.DMA,
        ],
    )
    def _scatter(x_hbm, pos_hbm, xs_hbm, idx_v, rows_v, sem):
        wid = lax.axis_index("s") * 2 + lax.axis_index("c")
        for half in range(TPW // CH):
            base = wid * TPW + half * CH
            pltpu.sync_copy(x_hbm.at[pl.ds(base, CH)], rows_v)
            for k in range(2):
                pltpu.sync_copy(pos_hbm.at[pl.ds(k * T + base, CH)], idx_v)
                pltpu.async_copy(rows_v, xs_hbm.at[idx_v], sem).wait()

    return _scatter


def _scatter_sc(x_flat, pos_flat):
    return _make_scatter()(x_flat, pos_flat)


def _gmm_body(bexp_ref, xs_ref, w1_ref, w2_ref, ys_ref):
    xb = xs_ref[...].astype(jnp.bfloat16)
    h = lax.dot_general(xb, w1_ref[0].astype(jnp.bfloat16),
                        (((1,), (1,)), ((), ())),
                        preferred_element_type=jnp.float32)
    h = h * jax.nn.sigmoid(h)
    y = lax.dot_general(h.astype(jnp.bfloat16),
                        w2_ref[0].astype(jnp.bfloat16),
                        (((1,), (1,)), ((), ())),
                        preferred_element_type=jnp.float32)
    ys_ref[...] = y


def _gmm(bexp, xs, w1, w2):
    grid_spec = pltpu.PrefetchScalarGridSpec(
        num_scalar_prefetch=1,
        grid=(NB,),
        in_specs=[
            pl.BlockSpec((BS, D), lambda b, be: (b, 0)),
            pl.BlockSpec((1, H, D), lambda b, be: (be[b], 0, 0)),
            pl.BlockSpec((1, D, H), lambda b, be: (be[b], 0, 0)),
        ],
        out_specs=pl.BlockSpec((BS, D), lambda b, be: (b, 0)),
    )
    return pl.pallas_call(
        _gmm_body,
        grid_spec=grid_spec,
        out_shape=jax.ShapeDtypeStruct((NP, D), jnp.float32),
    )(bexp, xs, w1, w2)


@functools.lru_cache(maxsize=None)
def _make_gather():
    mesh = plsc.VectorSubcoreMesh(core_axis_name="c", subcore_axis_name="s")

    @functools.partial(
        pl.kernel, mesh=mesh,
        out_type=(jax.ShapeDtypeStruct((T, D), jnp.float32),
                  jax.ShapeDtypeStruct((T, D), jnp.float32)),
        scratch_types=[
            pltpu.VMEM((CH,), jnp.int32),
            pltpu.VMEM((CH, D), jnp.float32),
            pltpu.SemaphoreType.DMA,
        ],
    )
    def _gather(ys_hbm, pos_hbm, b0_hbm, b1_hbm, idx_v, rows_v, sem):
        wid = lax.axis_index("s") * 2 + lax.axis_index("c")
        for half in range(TPW // CH):
            base = wid * TPW + half * CH
            for k, dst in ((0, b0_hbm), (1, b1_hbm)):
                pltpu.sync_copy(pos_hbm.at[pl.ds(k * T + base, CH)], idx_v)
                pltpu.async_copy(ys_hbm.at[idx_v], rows_v, sem).wait()
                pltpu.sync_copy(rows_v, dst.at[pl.ds(base, CH)])

    return _gather


def _gather_sc(ys, pos_flat):
    return _make_gather()(ys, pos_flat)


BLN = 512


def _ln_body(b0_ref, b1_ref, c0_ref, c1_ref, gamma_ref, beta_ref, o_ref):
    y = c0_ref[...] * b0_ref[...] + c1_ref[...] * b1_ref[...]
    mean = jnp.mean(y, axis=-1, keepdims=True)
    var = jnp.mean((y - mean) ** 2, axis=-1, keepdims=True)
    normed = (y - mean) * lax.rsqrt(var + 1e-5)
    o_ref[...] = normed * gamma_ref[...][None, :] + beta_ref[...][None, :]


def _combine_ln(b0, b1, c0, c1, gamma, beta):
    return pl.pallas_call(
        _ln_body,
        grid=(T // BLN,),
        in_specs=[
            pl.BlockSpec((BLN, D), lambda i: (i, 0)),
            pl.BlockSpec((BLN, D), lambda i: (i, 0)),
            pl.BlockSpec((BLN, 1), lambda i: (i, 0)),
            pl.BlockSpec((BLN, 1), lambda i: (i, 0)),
            pl.BlockSpec((D,), lambda i: (0,)),
            pl.BlockSpec((D,), lambda i: (0,)),
        ],
        out_specs=pl.BlockSpec((BLN, D), lambda i: (i, 0)),
        out_shape=jax.ShapeDtypeStruct((T, D), jnp.float32),
    )(b0, b1, c0, c1, gamma, beta)


def kernel(x, Wr, w1, w2, gamma, beta):
    B, S, Dm = x.shape
    x_flat = x.reshape(-1, Dm)
    cw, pos, bexp = _router(x_flat, Wr)
    pos_flat = pos.reshape(N)
    bexp_flat = bexp.reshape(64)[:NB]
    xs = _scatter_sc(x_flat, pos_flat)
    ys = _gmm(bexp_flat, xs, w1, w2)
    b0, b1 = _gather_sc(ys, pos_flat)
    out = _combine_ln(b0, b1, cw[:T], cw[T:], gamma, beta)
    return (out.reshape(B, S, Dm), jnp.asarray(0.0, dtype=jnp.float32))


# speedup vs baseline: 1.2264x; 1.2264x over previous
import functools

import jax
import jax.numpy as jnp
from jax import lax
from jax.experimental import pallas as pl
from jax.experimental.pallas import tpu as pltpu
from jax.experimental.pallas import tpu_sc as plsc

E = 8
D = 1024
DP = D // 2
H = 512
T = 4096
N = 2 * T
BS = 256
NB = N // BS + E - 1
NP = NB * BS
NW = 32
TPW = T // NW
CH = 128


def _pack_bf16(v):
    k = v.shape[-1] // 2
    u = pltpu.pack_elementwise([v[:, :k], v[:, k:]],
                               packed_dtype=jnp.bfloat16)
    return pltpu.bitcast(u, jnp.int32)


def _unpack_bf16(v):
    u = pltpu.bitcast(v, jnp.uint32)
    lo = pltpu.unpack_elementwise(u, index=0, packed_dtype=jnp.bfloat16,
                                  unpacked_dtype=jnp.float32)
    hi = pltpu.unpack_elementwise(u, index=1, packed_dtype=jnp.bfloat16,
                                  unpacked_dtype=jnp.float32)
    return jnp.concatenate([lo, hi], axis=1)


def _router_body(x_ref, wr_ref, xp_ref, c0_ref, c1_ref, pos_ref, bexp_ref,
                 a_ref, r_ref, g_ref, gx_ref):
    x = x_ref[...]
    xp_ref[...] = _pack_bf16(x)
    logits = lax.dot_general(x, wr_ref[...], (((1,), (1,)), ((), ())),
                             preferred_element_type=jnp.float32)
    iota_e = lax.broadcasted_iota(jnp.int32, (T, E), 1)
    l0 = jnp.max(logits, axis=1, keepdims=True)
    e0 = jnp.min(jnp.where(logits == l0, iota_e, E), axis=1, keepdims=True)
    masked = jnp.where(iota_e == e0, -jnp.inf, logits)
    l1 = jnp.max(masked, axis=1, keepdims=True)
    e1 = jnp.min(jnp.where(masked == l1, iota_e, E), axis=1, keepdims=True)
    c0 = jax.nn.sigmoid(l0 - l1)
    c0_ref[...] = c0
    c1_ref[...] = 1.0 - c0
    a_ref[0:T, :] = (iota_e == e0).astype(jnp.float32)
    a_ref[T:N, :] = (iota_e == e1).astype(jnp.float32)

    tri = (lax.broadcasted_iota(jnp.int32, (128, 128), 0) >=
           lax.broadcasted_iota(jnp.int32, (128, 128), 1)
           ).astype(jnp.float32)
    ngrp = N // 128

    def loop1(g, _):
        blk = a_ref[pl.ds(g * 128, 128), :]
        pre = lax.dot_general(tri, blk, (((1,), (0,)), ((), ())),
                              preferred_element_type=jnp.float32)
        r_ref[pl.ds(g * 128, 128), :] = pre
        g_ref[pl.ds(g, 1), :] = pre[127:128, :]
        return 0
    lax.fori_loop(0, ngrp, loop1, 0)

    gmat = g_ref[...]
    e64 = (lax.broadcasted_iota(jnp.int32, (ngrp, ngrp), 1) <
           lax.broadcasted_iota(jnp.int32, (ngrp, ngrp), 0)
           ).astype(jnp.float32)
    gx_ref[...] = lax.dot_general(e64, gmat, (((1,), (0,)), ((), ())),
                                  preferred_element_type=jnp.float32)
    counts = jnp.sum(gmat, axis=0, keepdims=True)
    pad = ((counts.astype(jnp.int32) + BS - 1) // BS) * BS
    u = (lax.broadcasted_iota(jnp.int32, (E, E), 0) <
         lax.broadcasted_iota(jnp.int32, (E, E), 1)).astype(jnp.float32)
    off = lax.dot_general(pad.astype(jnp.float32), u,
                          (((1,), (0,)), ((), ())),
                          preferred_element_type=jnp.float32)
    total = jnp.sum(pad, axis=1, keepdims=True)

    def loop2(g, _):
        ab = a_ref[pl.ds(g * 128, 128), :]
        rb = r_ref[pl.ds(g * 128, 128), :]
        gx = gx_ref[pl.ds(g, 1), :]
        pm = ab * (rb - 1.0 + gx + off)
        p = jnp.sum(pm, axis=1, keepdims=True)
        pos_ref[pl.ds(g * 128, 128), :] = p.astype(jnp.int32)
        return 0
    lax.fori_loop(0, ngrp, loop2, 0)

    pst = lax.broadcasted_iota(jnp.int32, (64, 1), 0) * BS
    be = jnp.sum((off <= pst.astype(jnp.float32)).astype(jnp.int32),
                 axis=1, keepdims=True) - 1
    be = jnp.where(pst < total, be, E)
    bexp_ref[...] = be[:NB, :]


def _router(x_flat, Wr):
    return pl.pallas_call(
        _router_body,
        grid=(1,),
        in_specs=[
            pl.BlockSpec((T, D), lambda i: (0, 0)),
            pl.BlockSpec((E, D), lambda i: (0, 0)),
        ],
        out_specs=[
            pl.BlockSpec((T, DP), lambda i: (0, 0)),
            pl.BlockSpec((T, 1), lambda i: (0, 0)),
            pl.BlockSpec((T, 1), lambda i: (0, 0)),
            pl.BlockSpec((N, 1), lambda i: (0, 0)),
            pl.BlockSpec((NB, 1), lambda i: (0, 0)),
        ],
        out_shape=[
            jax.ShapeDtypeStruct((T, DP), jnp.int32),
            jax.ShapeDtypeStruct((T, 1), jnp.float32),
            jax.ShapeDtypeStruct((T, 1), jnp.float32),
            jax.ShapeDtypeStruct((N, 1), jnp.int32),
            jax.ShapeDtypeStruct((NB, 1), jnp.int32),
        ],
        scratch_shapes=[
            pltpu.VMEM((N, E), jnp.float32),
            pltpu.VMEM((N, E), jnp.float32),
            pltpu.VMEM((N // 128, E), jnp.float32),
            pltpu.VMEM((N // 128, E), jnp.float32),
        ],
    )(x_flat, Wr)


@functools.lru_cache(maxsize=None)
def _make_scatter():
    mesh = plsc.VectorSubcoreMesh(core_axis_name="c", subcore_axis_name="s")

    @functools.partial(
        pl.kernel, mesh=mesh,
        out_type=jax.ShapeDtypeStruct((NP, DP), jnp.int32),
        scratch_types=[
            pltpu.VMEM((CH,), jnp.int32),
            pltpu.VMEM((CH, DP), jnp.int32),
            pltpu.SemaphoreType.DMA,
        ],
    )
    def _scatter(xp_hbm, pos_hbm, xs_hbm, idx_v, rows_v, sem):
        wid = lax.axis_index("s") * 2 + lax.axis_index("c")
        for half in range(TPW // CH):
            base = wid * TPW + half * CH
            pltpu.sync_copy(xp_hbm.at[pl.ds(base, CH)], rows_v)
            for k in range(2):
                pltpu.sync_copy(pos_hbm.at[pl.ds(k * T + base, CH)], idx_v)
                pltpu.async_copy(rows_v, xs_hbm.at[idx_v], sem).wait()

    return _scatter


def _scatter_sc(xp, pos_flat):
    return _make_scatter()(xp, pos_flat)


def _gmm_body(bexp_ref, xs_ref, w1_ref, w2_ref, ys_ref):
    @pl.when(bexp_ref[pl.program_id(0), 0] < E)
    def _():
        xb = _unpack_bf16(xs_ref[...]).astype(jnp.bfloat16)
        h = lax.dot_general(xb, w1_ref[0].astype(jnp.bfloat16),
                            (((1,), (1,)), ((), ())),
                            preferred_element_type=jnp.float32)
        h = h * jax.nn.sigmoid(h)
        y = lax.dot_general(h.astype(jnp.bfloat16),
                            w2_ref[0].astype(jnp.bfloat16),
                            (((1,), (1,)), ((), ())),
                            preferred_element_type=jnp.float32)
        ys_ref[...] = _pack_bf16(y)


def _gmm(bexp, xs, w1, w2):
    grid_spec = pltpu.PrefetchScalarGridSpec(
        num_scalar_prefetch=1,
        grid=(NB,),
        in_specs=[
            pl.BlockSpec((BS, DP), lambda b, be: (b, 0)),
            pl.BlockSpec((1, H, D),
                         lambda b, be: (jnp.minimum(be[b, 0], E - 1), 0, 0)),
            pl.BlockSpec((1, D, H),
                         lambda b, be: (jnp.minimum(be[b, 0], E - 1), 0, 0)),
        ],
        out_specs=pl.BlockSpec((BS, DP), lambda b, be: (b, 0)),
    )
    return pl.pallas_call(
        _gmm_body,
        grid_spec=grid_spec,
        out_shape=jax.ShapeDtypeStruct((NP, DP), jnp.int32),
    )(bexp, xs, w1, w2)


@functools.lru_cache(maxsize=None)
def _make_gather():
    mesh = plsc.VectorSubcoreMesh(core_axis_name="c", subcore_axis_name="s")

    @functools.partial(
        pl.kernel, mesh=mesh,
        out_type=(jax.ShapeDtypeStruct((T, DP), jnp.int32),
                  jax.ShapeDtypeStruct((T, DP), jnp.int32)),
        scratch_types=[
            pltpu.VMEM((CH,), jnp.int32),
            pltpu.VMEM((CH, DP), jnp.int32),
            pltpu.SemaphoreType.DMA,
        ],
    )
    def _gather(ys_hbm, pos_hbm, b0_hbm, b1_hbm, idx_v, rows_v, sem):
        wid = lax.axis_index("s") * 2 + lax.axis_index("c")
        for half in range(TPW // CH):
            base = wid * TPW + half * CH
            for k, dst in ((0, b0_hbm), (1, b1_hbm)):
                pltpu.sync_copy(pos_hbm.at[pl.ds(k * T + base, CH)], idx_v)
                pltpu.async_copy(ys_hbm.at[idx_v], rows_v, sem).wait()
                pltpu.sync_copy(rows_v, dst.at[pl.ds(base, CH)])

    return _gather


def _gather_sc(ys, pos_flat):
    return _make_gather()(ys, pos_flat)


BLN = 512


def _ln_body(b0_ref, b1_ref, c0_ref, c1_ref, gamma_ref, beta_ref, o_ref):
    r0 = _unpack_bf16(b0_ref[...])
    r1 = _unpack_bf16(b1_ref[...])
    y = c0_ref[...] * r0 + c1_ref[...] * r1
    mean = jnp.mean(y, axis=-1, keepdims=True)
    var = jnp.mean((y - mean) ** 2, axis=-1, keepdims=True)
    normed = (y - mean) * lax.rsqrt(var + 1e-5)
    o_ref[...] = normed * gamma_ref[...][None, :] + beta_ref[...][None, :]


def _combine_ln(b0, b1, c0, c1, gamma, beta):
    return pl.pallas_call(
        _ln_body,
        grid=(T // BLN,),
        in_specs=[
            pl.BlockSpec((BLN, DP), lambda i: (i, 0)),
            pl.BlockSpec((BLN, DP), lambda i: (i, 0)),
            pl.BlockSpec((BLN, 1), lambda i: (i, 0)),
            pl.BlockSpec((BLN, 1), lambda i: (i, 0)),
            pl.BlockSpec((D,), lambda i: (0,)),
            pl.BlockSpec((D,), lambda i: (0,)),
        ],
        out_specs=pl.BlockSpec((BLN, D), lambda i: (i, 0)),
        out_shape=jax.ShapeDtypeStruct((T, D), jnp.float32),
    )(b0, b1, c0, c1, gamma, beta)


def kernel(x, Wr, w1, w2, gamma, beta):
    B, S, Dm = x.shape
    x_flat = x.reshape(-1, Dm)
    xp, c0, c1, pos, bexp = _router(x_flat, Wr)
    pos_flat = pos.reshape(N)
    xs = _scatter_sc(xp, pos_flat)
    ys = _gmm(bexp, xs, w1, w2)
    b0, b1 = _gather_sc(ys, pos_flat)
    out = _combine_ln(b0, b1, c0, c1, gamma, beta)
    return (out.reshape(B, S, Dm), jnp.asarray(0.0, dtype=jnp.float32))


# speedup vs baseline: 1.2333x; 1.0056x over previous
import functools

import jax
import jax.numpy as jnp
from jax import lax
from jax.experimental import pallas as pl
from jax.experimental.pallas import tpu as pltpu
from jax.experimental.pallas import tpu_sc as plsc

E = 8
D = 1024
DP = D // 2
H = 512
T = 4096
N = 2 * T
BS = 256
NB = N // BS + E - 1
NP = NB * BS
NW = 32
TPW = T // NW
CH = 128


def _pack_bf16(v):
    k = v.shape[-1] // 2
    u = pltpu.pack_elementwise([v[:, :k], v[:, k:]],
                               packed_dtype=jnp.bfloat16)
    return pltpu.bitcast(u, jnp.int32)


def _unpack_bf16(v):
    u = pltpu.bitcast(v, jnp.uint32)
    lo = pltpu.unpack_elementwise(u, index=0, packed_dtype=jnp.bfloat16,
                                  unpacked_dtype=jnp.float32)
    hi = pltpu.unpack_elementwise(u, index=1, packed_dtype=jnp.bfloat16,
                                  unpacked_dtype=jnp.float32)
    return jnp.concatenate([lo, hi], axis=1)


def _router_body(x_ref, wr_ref, xp_ref, c0_ref, c1_ref, pos_ref, bexp_ref,
                 a_ref, r_ref, g_ref, gx_ref):
    x = x_ref[...]
    xp_ref[...] = _pack_bf16(x)
    logits = lax.dot_general(x, wr_ref[...], (((1,), (1,)), ((), ())),
                             preferred_element_type=jnp.float32)
    iota_e = lax.broadcasted_iota(jnp.int32, (T, E), 1)
    l0 = jnp.max(logits, axis=1, keepdims=True)
    e0 = jnp.min(jnp.where(logits == l0, iota_e, E), axis=1, keepdims=True)
    masked = jnp.where(iota_e == e0, -jnp.inf, logits)
    l1 = jnp.max(masked, axis=1, keepdims=True)
    e1 = jnp.min(jnp.where(masked == l1, iota_e, E), axis=1, keepdims=True)
    c0 = jax.nn.sigmoid(l0 - l1)
    c0_ref[...] = c0
    c1_ref[...] = 1.0 - c0
    a_ref[0:T, :] = (iota_e == e0).astype(jnp.float32)
    a_ref[T:N, :] = (iota_e == e1).astype(jnp.float32)

    tri = (lax.broadcasted_iota(jnp.int32, (128, 128), 0) >=
           lax.broadcasted_iota(jnp.int32, (128, 128), 1)
           ).astype(jnp.float32)
    ngrp = N // 128

    def loop1(g, _):
        blk = a_ref[pl.ds(g * 128, 128), :]
        pre = lax.dot_general(tri, blk, (((1,), (0,)), ((), ())),
                              preferred_element_type=jnp.float32)
        r_ref[pl.ds(g * 128, 128), :] = pre
        g_ref[pl.ds(g, 1), :] = pre[127:128, :]
        return 0
    lax.fori_loop(0, ngrp, loop1, 0)

    gmat = g_ref[...]
    e64 = (lax.broadcasted_iota(jnp.int32, (ngrp, ngrp), 1) <
           lax.broadcasted_iota(jnp.int32, (ngrp, ngrp), 0)
           ).astype(jnp.float32)
    gx_ref[...] = lax.dot_general(e64, gmat, (((1,), (0,)), ((), ())),
                                  preferred_element_type=jnp.float32)
    counts = jnp.sum(gmat, axis=0, keepdims=True)
    pad = ((counts.astype(jnp.int32) + BS - 1) // BS) * BS
    u = (lax.broadcasted_iota(jnp.int32, (E, E), 0) <
         lax.broadcasted_iota(jnp.int32, (E, E), 1)).astype(jnp.float32)
    off = lax.dot_general(pad.astype(jnp.float32), u,
                          (((1,), (0,)), ((), ())),
                          preferred_element_type=jnp.float32)
    total = jnp.sum(pad, axis=1, keepdims=True)

    def loop2(g, _):
        ab = a_ref[pl.ds(g * 128, 128), :]
        rb = r_ref[pl.ds(g * 128, 128), :]
        gx = gx_ref[pl.ds(g, 1), :]
        pm = ab * (rb - 1.0 + gx + off)
        p = jnp.sum(pm, axis=1, keepdims=True)
        pos_ref[pl.ds(g * 128, 128), :] = p.astype(jnp.int32)
        return 0
    lax.fori_loop(0, ngrp, loop2, 0)

    pst = lax.broadcasted_iota(jnp.int32, (64, 1), 0) * BS
    be = jnp.sum((off <= pst.astype(jnp.float32)).astype(jnp.int32),
                 axis=1, keepdims=True) - 1
    be = jnp.where(pst < total, be, E)
    bexp_ref[...] = be[:NB, :]


def _router(x_flat, Wr):
    return pl.pallas_call(
        _router_body,
        grid=(1,),
        in_specs=[
            pl.BlockSpec((T, D), lambda i: (0, 0)),
            pl.BlockSpec((E, D), lambda i: (0, 0)),
        ],
        out_specs=[
            pl.BlockSpec((T, DP), lambda i: (0, 0)),
            pl.BlockSpec((T, 1), lambda i: (0, 0)),
            pl.BlockSpec((T, 1), lambda i: (0, 0)),
            pl.BlockSpec((N, 1), lambda i: (0, 0)),
            pl.BlockSpec((NB, 1), lambda i: (0, 0)),
        ],
        out_shape=[
            jax.ShapeDtypeStruct((T, DP), jnp.int32),
            jax.ShapeDtypeStruct((T, 1), jnp.float32),
            jax.ShapeDtypeStruct((T, 1), jnp.float32),
            jax.ShapeDtypeStruct((N, 1), jnp.int32),
            jax.ShapeDtypeStruct((NB, 1), jnp.int32),
        ],
        scratch_shapes=[
            pltpu.VMEM((N, E), jnp.float32),
            pltpu.VMEM((N, E), jnp.float32),
            pltpu.VMEM((N // 128, E), jnp.float32),
            pltpu.VMEM((N // 128, E), jnp.float32),
        ],
    )(x_flat, Wr)


@functools.lru_cache(maxsize=None)
def _make_scatter():
    mesh = plsc.VectorSubcoreMesh(core_axis_name="c", subcore_axis_name="s")

    @functools.partial(
        pl.kernel, mesh=mesh,
        out_type=jax.ShapeDtypeStruct((NP, DP), jnp.int32),
        scratch_types=[
            pltpu.VMEM((CH,), jnp.int32),
            pltpu.VMEM((CH, DP), jnp.int32),
            pltpu.SemaphoreType.DMA,
        ],
    )
    def _scatter(xp_hbm, pos_hbm, xs_hbm, idx_v, rows_v, sem):
        wid = lax.axis_index("s") * 2 + lax.axis_index("c")
        for half in range(TPW // CH):
            base = wid * TPW + half * CH
            pltpu.sync_copy(xp_hbm.at[pl.ds(base, CH)], rows_v)
            for k in range(2):
                pltpu.sync_copy(pos_hbm.at[pl.ds(k * T + base, CH)], idx_v)
                pltpu.async_copy(rows_v, xs_hbm.at[idx_v], sem).wait()

    return _scatter


def _scatter_sc(xp, pos_flat):
    return _make_scatter()(xp, pos_flat)


def _gmm_body(bexp_ref, xs_ref, w1_ref, w2_ref, ys_ref):
    @pl.when(bexp_ref[pl.program_id(0), 0] < E)
    def _():
        xb = _unpack_bf16(xs_ref[...])
        h = lax.dot_general(xb, w1_ref[0],
                            (((1,), (1,)), ((), ())),
                            preferred_element_type=jnp.float32)
        h = h * jax.nn.sigmoid(h)
        y = lax.dot_general(h, w2_ref[0],
                            (((1,), (1,)), ((), ())),
                            preferred_element_type=jnp.float32)
        ys_ref[...] = _pack_bf16(y)


def _gmm(bexp, xs, w1, w2):
    grid_spec = pltpu.PrefetchScalarGridSpec(
        num_scalar_prefetch=1,
        grid=(NB,),
        in_specs=[
            pl.BlockSpec((BS, DP), lambda b, be: (b, 0)),
            pl.BlockSpec((1, H, D),
                         lambda b, be: (jnp.minimum(be[b, 0], E - 1), 0, 0)),
            pl.BlockSpec((1, D, H),
                         lambda b, be: (jnp.minimum(be[b, 0], E - 1), 0, 0)),
        ],
        out_specs=pl.BlockSpec((BS, DP), lambda b, be: (b, 0)),
    )
    return pl.pallas_call(
        _gmm_body,
        grid_spec=grid_spec,
        out_shape=jax.ShapeDtypeStruct((NP, DP), jnp.int32),
    )(bexp, xs, w1, w2)


@functools.lru_cache(maxsize=None)
def _make_gather():
    mesh = plsc.VectorSubcoreMesh(core_axis_name="c", subcore_axis_name="s")

    @functools.partial(
        pl.kernel, mesh=mesh,
        out_type=(jax.ShapeDtypeStruct((T, DP), jnp.int32),
                  jax.ShapeDtypeStruct((T, DP), jnp.int32)),
        scratch_types=[
            pltpu.VMEM((CH,), jnp.int32),
            pltpu.VMEM((CH, DP), jnp.int32),
            pltpu.SemaphoreType.DMA,
        ],
    )
    def _gather(ys_hbm, pos_hbm, b0_hbm, b1_hbm, idx_v, rows_v, sem):
        wid = lax.axis_index("s") * 2 + lax.axis_index("c")
        for half in range(TPW // CH):
            base = wid * TPW + half * CH
            for k, dst in ((0, b0_hbm), (1, b1_hbm)):
                pltpu.sync_copy(pos_hbm.at[pl.ds(k * T + base, CH)], idx_v)
                pltpu.async_copy(ys_hbm.at[idx_v], rows_v, sem).wait()
                pltpu.sync_copy(rows_v, dst.at[pl.ds(base, CH)])

    return _gather


def _gather_sc(ys, pos_flat):
    return _make_gather()(ys, pos_flat)


BLN = 512


def _ln_body(b0_ref, b1_ref, c0_ref, c1_ref, gamma_ref, beta_ref, o_ref):
    r0 = _unpack_bf16(b0_ref[...])
    r1 = _unpack_bf16(b1_ref[...])
    y = c0_ref[...] * r0 + c1_ref[...] * r1
    mean = jnp.mean(y, axis=-1, keepdims=True)
    var = jnp.mean((y - mean) ** 2, axis=-1, keepdims=True)
    normed = (y - mean) * lax.rsqrt(var + 1e-5)
    o_ref[...] = normed * gamma_ref[...][None, :] + beta_ref[...][None, :]


def _combine_ln(b0, b1, c0, c1, gamma, beta):
    return pl.pallas_call(
        _ln_body,
        grid=(T // BLN,),
        in_specs=[
            pl.BlockSpec((BLN, DP), lambda i: (i, 0)),
            pl.BlockSpec((BLN, DP), lambda i: (i, 0)),
            pl.BlockSpec((BLN, 1), lambda i: (i, 0)),
            pl.BlockSpec((BLN, 1), lambda i: (i, 0)),
            pl.BlockSpec((D,), lambda i: (0,)),
            pl.BlockSpec((D,), lambda i: (0,)),
        ],
        out_specs=pl.BlockSpec((BLN, D), lambda i: (i, 0)),
        out_shape=jax.ShapeDtypeStruct((T, D), jnp.float32),
    )(b0, b1, c0, c1, gamma, beta)


def kernel(x, Wr, w1, w2, gamma, beta):
    B, S, Dm = x.shape
    x_flat = x.reshape(-1, Dm)
    xp, c0, c1, pos, bexp = _router(x_flat, Wr)
    pos_flat = pos.reshape(N)
    xs = _scatter_sc(xp, pos_flat)
    ys = _gmm(bexp, xs, w1, w2)
    b0, b1 = _gather_sc(ys, pos_flat)
    out = _combine_ln(b0, b1, c0, c1, gamma, beta)
    return (out.reshape(B, S, Dm), jnp.asarray(0.0, dtype=jnp.float32))
